# baseline (speedup 1.0000x reference)
import functools

import jax
import jax.numpy as jnp
from jax import lax
from jax.experimental import pallas as pl
from jax.experimental.pallas import tpu as pltpu
from jax.experimental.pallas import tpu_sc as plsc

N = 10000
E = 320000
NC = 2
NS = 16
NW = NC * NS
K = 128
NCHUNK = 80
EPT = NCHUNK * K
E_PAD = NW * EPT - E
NBUF = 2
RPT = 624
TAIL = N - NS * RPT
L = 16
BN = 1000


def _sc_mesh():
    return plsc.VectorSubcoreMesh(core_axis_name="c", subcore_axis_name="s")


def _stage_edges(src_hbm, dst_hbm, w_hbm, srcm, dstm, wm, gwid):
    base = gwid * NCHUNK
    pltpu.sync_copy(src_hbm.at[pl.ds(base, NCHUNK)], srcm)
    pltpu.sync_copy(dst_hbm.at[pl.ds(base, NCHUNK)], dstm)
    pltpu.sync_copy(w_hbm.at[pl.ds(base, NCHUNK)], wm)

    def mask_chunk(i, carry):
        for g in range(K // L):
            sl = pl.ds(g * L, L)
            s16 = srcm[i, sl]
            d16 = dstm[i, sl]
            w16 = wm[i, sl]
            wm[i, sl] = jnp.where(s16 == d16, jnp.zeros((L,), jnp.float32), w16)
        return carry

    lax.fori_loop(0, NCHUNK, mask_chunk, 0)


def _rowwise_copy(copy_fn, sid):
    copy_fn(pl.ds(sid * RPT, RPT))

    @pl.when(sid == NS - 1)
    def _():
        copy_fn(pl.ds(NS * RPT, TAIL))


def _make_sc_edge_pass(C):
    KC = 16 if C == 128 else K
    TOTCH = (NW * EPT) // KC
    if C == 128:
        NB, LEAD = 4, 2
        CNT0, CNT1 = 880, 400
    elif C == 64:
        NB, LEAD = 5, 3
        CNT0, CNT1 = 105, 55
    else:
        NB, LEAD = 8, 6
        CNT0, CNT1 = 112, 48

    def body(y_hbm, src_hbm, dst_hbm, w_hbm, z_hbm, out_hbm,
             acc, srcm, dstm, wm, rows, *sems):
        gsems = sems[:NB]
        ssems = sems[NB:]
        cid = lax.axis_index("c")
        sid = lax.axis_index("s")
        _rowwise_copy(lambda sl: pltpu.sync_copy(z_hbm.at[sl], acc.at[sl]), sid)
        start = jnp.where(cid == 0, sid * CNT0, NS * CNT0 + sid * CNT1)
        nch = jnp.where(cid == 0, CNT0, CNT1)

        @pl.when(cid == 0)
        def _():
            pltpu.sync_copy(src_hbm.at[pl.ds(start, CNT0)], srcm)
            pltpu.sync_copy(dst_hbm.at[pl.ds(start, CNT0)], dstm)
            pltpu.sync_copy(w_hbm.at[pl.ds(start, CNT0)], wm)

        @pl.when(cid == 1)
        def _():
            csl = pl.ds(0, CNT1)
            pltpu.sync_copy(src_hbm.at[pl.ds(start, CNT1)], srcm.at[csl])
            pltpu.sync_copy(dst_hbm.at[pl.ds(start, CNT1)], dstm.at[csl])
            pltpu.sync_copy(w_hbm.at[pl.ds(start, CNT1)], wm.at[csl])

        def mask_chunk(i, carry):
            for g in range(KC // L):
                sl = pl.ds(g * L, L)
                wm[i, sl] = jnp.where(srcm[i, sl] == dstm[i, sl],
                                      jnp.zeros((L,), jnp.float32), wm[i, sl])
            return carry

        lax.fori_loop(0, nch, mask_chunk, 0)
        plsc.subcore_barrier()

        for c0 in range(LEAD):
            pltpu.async_copy(y_hbm.at[srcm.at[c0]], rows.at[c0], gsems[c0])

        def visit(cur, b):
            pltpu.make_async_copy(
                y_hbm.at[srcm.at[cur]], rows.at[b], gsems[b]).wait()

            def gbody(g, cc):
                w16 = wm[cur, pl.ds(g * L, L)]
                for j in range(L):
                    we = w16[j]
                    e = g * L + j
                    for ci in range(C // L):
                        csl = pl.ds(ci * L, L)
                        rows[b, e, csl] = rows[b, e, csl] * we
                return cc

            lax.fori_loop(0, KC // L, gbody, 0)
            pltpu.async_copy(rows.at[b], acc.at[dstm.at[cur]], ssems[b],
                             add=True)
            nxt = cur + LEAD
            bj = (b + LEAD) % NB
            prev = cur + LEAD - NB

            @pl.when(prev >= 0)
            def _():
                pltpu.make_async_copy(
                    rows.at[bj], acc.at[dstm.at[prev]], ssems[bj]).wait()

            @pl.when(nxt < nch)
            def _():
                pltpu.async_copy(y_hbm.at[srcm.at[nxt]], rows.at[bj],
                                 gsems[bj])

        def outer(i, carry):
            for b in range(NB):
                visit(i * NB + b, b)
            return carry

        lax.fori_loop(0, nch // NB, outer, 0)
        for t in range(NB - LEAD):
            bt = (LEAD + t) % NB
            pltpu.make_async_copy(
                rows.at[bt], acc.at[dstm.at[nch - (NB - LEAD) + t]],
                ssems[bt]).wait()
        plsc.subcore_barrier()
        _rowwise_copy(
            lambda sl: pltpu.sync_copy(acc.at[sl], out_hbm.at[cid, sl]), sid)

    return pl.kernel(
        body,
        out_type=jax.ShapeDtypeStruct((NC, N, C), jnp.float32),
        mesh=_sc_mesh(),
        compiler_params=pltpu.CompilerParams(use_tc_tiling_on_sc=False),
        scratch_types=[
            pltpu.VMEM_SHARED((N, C), jnp.float32),
            pltpu.VMEM((CNT0, KC), jnp.int32),
            pltpu.VMEM((CNT0, KC), jnp.int32),
            pltpu.VMEM((CNT0, KC), jnp.float32),
            pltpu.VMEM((NB, KC, C), jnp.float32),
        ] + [pltpu.SemaphoreType.DMA] * (2 * NB),
    )


def _make_sc_deg():
    def body(src_hbm, dst_hbm, w_hbm, z_hbm, out_hbm,
             acc, srcm, dstm, wm, rows, sem0, sem1):
        ssems = (sem0, sem1)
        cid = lax.axis_index("c")
        sid = lax.axis_index("s")
        gwid = cid * NS + sid
        _rowwise_copy(lambda sl: pltpu.sync_copy(z_hbm.at[sl], acc.at[sl]), sid)
        _stage_edges(src_hbm, dst_hbm, w_hbm, srcm, dstm, wm, gwid)
        plsc.subcore_barrier()

        def visit(cur, b):
            @pl.when(cur >= 2)
            def _():
                pltpu.make_async_copy(
                    rows.at[b], acc.at[srcm.at[cur - 2]], ssems[b]).wait()

            def gbody(g, cc):
                w16 = wm[cur, pl.ds(g * L, L)]
                for j in range(L):
                    e = g * L + j
                    rows[b, e, pl.ds(0, L)] = jnp.full((L,), w16[j],
                                                       jnp.float32)
                return cc

            lax.fori_loop(0, K // L, gbody, 0)
            pltpu.async_copy(rows.at[b], acc.at[srcm.at[cur]], ssems[b],
                             add=True)

        def outer(i, carry):
            for b in range(2):
                visit(i * 2 + b, b)
            return carry

        lax.fori_loop(0, NCHUNK // 2, outer, 0)
        for tail in range(NCHUNK - 2, NCHUNK):
            pltpu.make_async_copy(
                rows.at[tail % 2], acc.at[srcm.at[tail]],
                ssems[tail % 2]).wait()
        plsc.subcore_barrier()
        _rowwise_copy(
            lambda sl: pltpu.sync_copy(acc.at[sl], out_hbm.at[cid, sl]), sid)

    return pl.kernel(
        body,
        out_type=jax.ShapeDtypeStruct((NC, N, L), jnp.float32),
        mesh=_sc_mesh(),
        compiler_params=pltpu.CompilerParams(use_tc_tiling_on_sc=False),
        scratch_types=[
            pltpu.VMEM_SHARED((N, L), jnp.float32),
            pltpu.VMEM((NCHUNK, K), jnp.int32),
            pltpu.VMEM((NCHUNK, K), jnp.int32),
            pltpu.VMEM((NCHUNK, K), jnp.float32),
            pltpu.VMEM((2, K, L), jnp.float32),
            pltpu.SemaphoreType.DMA,
            pltpu.SemaphoreType.DMA,
        ],
    )


def _tc_pre(degp, x):
    def body(degp_ref, x_ref, dis_ref, a_ref):
        d = degp_ref[0] + degp_ref[1]
        pos = d > 0.0
        safe = jnp.where(pos, d, 1.0)
        dis = jnp.where(pos, lax.rsqrt(safe), 0.0)
        dis_ref[...] = dis
        a_ref[...] = x_ref[...] * dis[:, 0:1]

    return pl.pallas_call(
        body,
        grid=(N // BN,),
        in_specs=[
            pl.BlockSpec((2, BN, L), lambda i: (0, i, 0)),
            pl.BlockSpec((BN, 128), lambda i: (i, 0)),
        ],
        out_specs=[
            pl.BlockSpec((BN, L), lambda i: (i, 0)),
            pl.BlockSpec((BN, 128), lambda i: (i, 0)),
        ],
        out_shape=[
            jax.ShapeDtypeStruct((N, L), jnp.float32),
            jax.ShapeDtypeStruct((N, 128), jnp.float32),
        ],
    )(degp, x)


def _tc_mid(r1p, h, dis, W, Ci, Co):
    def body(r_ref, h_ref, dis_ref, w_ref, op_ref, tp_ref):
        d = dis_ref[:, 0:1]
        y1 = -d * (r_ref[0] + r_ref[1])
        v0 = w_ref[0] - w_ref[2]
        op = jnp.dot(h_ref[...], v0, preferred_element_type=jnp.float32)
        op = op + jnp.dot(y1, w_ref[1], preferred_element_type=jnp.float32)
        op_ref[...] = op
        tp_ref[...] = d * jnp.dot(y1, w_ref[2],
                                  preferred_element_type=jnp.float32)

    return pl.pallas_call(
        body,
        grid=(N // BN,),
        in_specs=[
            pl.BlockSpec((2, BN, Ci), lambda i: (0, i, 0)),
            pl.BlockSpec((BN, Ci), lambda i: (i, 0)),
            pl.BlockSpec((BN, L), lambda i: (i, 0)),
            pl.BlockSpec((3, Ci, Co), lambda i: (0, 0, 0)),
        ],
        out_specs=[
            pl.BlockSpec((BN, Co), lambda i: (i, 0)),
            pl.BlockSpec((BN, Co), lambda i: (i, 0)),
        ],
        out_shape=[
            jax.ShapeDtypeStruct((N, Co), jnp.float32),
            jax.ShapeDtypeStruct((N, Co), jnp.float32),
        ],
    )(r1p, h, dis, W)


def _tc_post(r2p, op, dis, b8, Co):
    def body(r_ref, op_ref, dis_ref, b_ref, h_ref, a_ref):
        d = dis_ref[:, 0:1]
        o = op_ref[...] + b_ref[0:1, :] - 2.0 * d * (r_ref[0] + r_ref[1])
        h = jnp.maximum(o, 0.0)
        h_ref[...] = h
        a_ref[...] = d * h

    return pl.pallas_call(
        body,
        grid=(N // BN,),
        in_specs=[
            pl.BlockSpec((2, BN, Co), lambda i: (0, i, 0)),
            pl.BlockSpec((BN, Co), lambda i: (i, 0)),
            pl.BlockSpec((BN, L), lambda i: (i, 0)),
            pl.BlockSpec((8, Co), lambda i: (0, 0)),
        ],
        out_specs=[
            pl.BlockSpec((BN, Co), lambda i: (i, 0)),
            pl.BlockSpec((BN, Co), lambda i: (i, 0)),
        ],
        out_shape=[
            jax.ShapeDtypeStruct((N, Co), jnp.float32),
            jax.ShapeDtypeStruct((N, Co), jnp.float32),
        ],
    )(r2p, op, dis, b8)


def _tc_final(r2p, op, dis, b8):
    def body(r_ref, op_ref, dis_ref, b_ref, out_ref):
        d = dis_ref[:, 0:1]
        o = op_ref[...] + b_ref[0:1, :] - 2.0 * d * (r_ref[0] + r_ref[1])
        z = o[:, :10]
        m = jnp.max(z, axis=1, keepdims=True)
        zs = z - m
        lse = jnp.log(jnp.sum(jnp.exp(zs), axis=1, keepdims=True))
        out_ref[...] = zs - lse

    return pl.pallas_call(
        body,
        grid=(N // BN,),
        in_specs=[
            pl.BlockSpec((2, BN, 16), lambda i: (0, i, 0)),
            pl.BlockSpec((BN, 16), lambda i: (i, 0)),
            pl.BlockSpec((BN, L), lambda i: (i, 0)),
            pl.BlockSpec((8, 16), lambda i: (0, 0)),
        ],
        out_specs=pl.BlockSpec((BN, 10), lambda i: (i, 0)),
        out_shape=jax.ShapeDtypeStruct((N, 10), jnp.float32),
    )(r2p, op, dis, b8)


_sc_pass = {C: _make_sc_edge_pass(C) for C in (128, 64, 32, 16)}
_sc_deg = _make_sc_deg()


def kernel(x, edge_index, edge_weight, W1, b1, W2, b2, W3, b3, W4, b4):
    zpad_i = jnp.zeros((E_PAD,), jnp.int32)
    srcf = jnp.concatenate([edge_index[0], zpad_i])
    dstf = jnp.concatenate([edge_index[1], zpad_i])
    wf = jnp.concatenate([edge_weight, jnp.zeros((E_PAD,), jnp.float32)])

    def eshape(C):
        kc = 16 if C == 128 else K
        return ((NW * EPT) // kc, kc)

    edges = {C: (srcf.reshape(eshape(C)), dstf.reshape(eshape(C)),
                 wf.reshape(eshape(C)))
             for C in (128, 64, 32, 16)}
    src3, dst3, w3 = edges[16]
    zeros = {C: jnp.zeros((N, C), jnp.float32) for C in (128, 64, 32, 16)}

    W4p = jnp.zeros((3, 16, 16), jnp.float32).at[:, :, :10].set(W4)
    b4p = jnp.zeros((16,), jnp.float32).at[:10].set(b4)

    degp = _sc_deg(src3, dst3, w3, zeros[16])
    dis, a = _tc_pre(degp, x)

    layers = [
        (W1, b1, 128, 64),
        (W2, b2, 64, 32),
        (W3, b3, 32, 16),
        (W4p, b4p, 16, 16),
    ]
    h = x
    for li, (W, b, Ci, Co) in enumerate(layers):
        b8 = jnp.broadcast_to(b, (8, Co))
        r1p = _sc_pass[Ci](a, *edges[Ci], zeros[Ci])
        op, tp = _tc_mid(r1p, h, dis, W, Ci, Co)
        r2p = _sc_pass[Co](tp, *edges[Co], zeros[Co])
        if li < 3:
            h, a = _tc_post(r2p, op, dis, b8, Co)
        else:
            return _tc_final(r2p, op, dis, b8)


# speedup vs baseline: 1.2054x; 1.2054x over previous
import functools

import jax
import jax.numpy as jnp
import numpy as np
from jax import lax
from jax.experimental import pallas as pl
from jax.experimental.pallas import tpu as pltpu
from jax.experimental.pallas import tpu_sc as plsc

N = 10000
E = 320000
NC = 2
NS = 16
NW = NC * NS
K = 128
NCHUNK = 80
EPT = NCHUNK * K
E_PAD = NW * EPT - E
NBUF = 2
RPT = 624
TAIL = N - NS * RPT
L = 16
BN = 1000


def _sc_mesh():
    return plsc.VectorSubcoreMesh(core_axis_name="c", subcore_axis_name="s")


def _stage_edges(src_hbm, dst_hbm, w_hbm, srcm, dstm, wm, gwid):
    base = gwid * NCHUNK
    pltpu.sync_copy(src_hbm.at[pl.ds(base, NCHUNK)], srcm)
    pltpu.sync_copy(dst_hbm.at[pl.ds(base, NCHUNK)], dstm)
    pltpu.sync_copy(w_hbm.at[pl.ds(base, NCHUNK)], wm)

    def mask_chunk(i, carry):
        for g in range(K // L):
            sl = pl.ds(g * L, L)
            s16 = srcm[i, sl]
            d16 = dstm[i, sl]
            w16 = wm[i, sl]
            wm[i, sl] = jnp.where(s16 == d16, jnp.zeros((L,), jnp.float32), w16)
        return carry

    lax.fori_loop(0, NCHUNK, mask_chunk, 0)


def _rowwise_copy(copy_fn, sid):
    copy_fn(pl.ds(sid * RPT, RPT))

    @pl.when(sid == NS - 1)
    def _():
        copy_fn(pl.ds(NS * RPT, TAIL))


def _make_sc_edge_pass(C):
    KC = 16 if C == 128 else K
    TOTCH = (NW * EPT) // KC
    NB, LEAD = 4, 2
    BF = C >= 32
    if C == 128:
        CNT0, CNT1 = 800, 480
    elif C == 64:
        CNT0, CNT1 = 104, 56
    else:
        CNT0, CNT1 = 112, 48

    def body(y_hbm, src_hbm, dst_hbm, w_hbm, z_hbm, out_hbm,
             acc, srcm, dstm, wm, rows, srow, *sems):
        gsems = sems[:NB]
        ssems = sems[NB:]
        cid = lax.axis_index("c")
        sid = lax.axis_index("s")
        _rowwise_copy(lambda sl: pltpu.sync_copy(z_hbm.at[sl], acc.at[sl]), sid)
        start = jnp.where(cid == 0, sid * CNT0, NS * CNT0 + sid * CNT1)
        nch = jnp.where(cid == 0, CNT0, CNT1)

        @pl.when(cid == 0)
        def _():
            pltpu.sync_copy(src_hbm.at[pl.ds(start, CNT0)], srcm)
            pltpu.sync_copy(dst_hbm.at[pl.ds(start, CNT0)], dstm)
            pltpu.sync_copy(w_hbm.at[pl.ds(start, CNT0)], wm)

        @pl.when(cid == 1)
        def _():
            csl = pl.ds(0, CNT1)
            pltpu.sync_copy(src_hbm.at[pl.ds(start, CNT1)], srcm.at[csl])
            pltpu.sync_copy(dst_hbm.at[pl.ds(start, CNT1)], dstm.at[csl])
            pltpu.sync_copy(w_hbm.at[pl.ds(start, CNT1)], wm.at[csl])

        def mask_chunk(i, carry):
            for g in range(KC // L):
                sl = pl.ds(g * L, L)
                wm[i, sl] = jnp.where(srcm[i, sl] == dstm[i, sl],
                                      jnp.zeros((L,), jnp.float32), wm[i, sl])
            return carry

        lax.fori_loop(0, nch, mask_chunk, 0)
        plsc.subcore_barrier()

        for c0 in range(LEAD):
            pltpu.async_copy(y_hbm.at[srcm.at[c0]], rows.at[c0], gsems[c0])

        def visit(cur, b):
            pltpu.make_async_copy(
                y_hbm.at[srcm.at[cur]], rows.at[b], gsems[b]).wait()
            prev = cur - NB

            @pl.when(prev >= 0)
            def _():
                pltpu.make_async_copy(
                    srow.at[b], acc.at[dstm.at[prev]], ssems[b]).wait()

            def gbody(g, cc):
                w16 = wm[cur, pl.ds(g * L, L)]
                for j in range(L):
                    we = w16[j]
                    e = g * L + j
                    if BF:
                        for ci in range(C // (2 * L)):
                            v = rows[b, e, pl.ds(ci * 2 * L, 2 * L)]
                            lo, hi = plsc.unpack(
                                v, format=plsc.PackFormat.INTERLEAVED)
                            srow[b, e, pl.ds(ci * L, L)] = lo * we
                            srow[b, e, pl.ds(C // 2 + ci * L, L)] = hi * we
                    else:
                        for ci in range(C // L):
                            csl = pl.ds(ci * L, L)
                            srow[b, e, csl] = rows[b, e, csl] * we
                return cc

            lax.fori_loop(0, KC // L, gbody, 0)
            pltpu.async_copy(srow.at[b], acc.at[dstm.at[cur]], ssems[b],
                             add=True)
            nxt = cur + LEAD
            bj = (b + LEAD) % NB

            @pl.when(nxt < nch)
            def _():
                pltpu.async_copy(y_hbm.at[srcm.at[nxt]], rows.at[bj],
                                 gsems[bj])

        def outer(i, carry):
            for b in range(NB):
                visit(i * NB + b, b)
            return carry

        lax.fori_loop(0, nch // NB, outer, 0)
        for t in range(NB):
            pltpu.make_async_copy(
                srow.at[t], acc.at[dstm.at[nch - NB + t]],
                ssems[t]).wait()
        plsc.subcore_barrier()
        _rowwise_copy(
            lambda sl: pltpu.sync_copy(acc.at[sl], out_hbm.at[cid, sl]), sid)

    return pl.kernel(
        body,
        out_type=jax.ShapeDtypeStruct((NC, N, C), jnp.float32),
        mesh=_sc_mesh(),
        compiler_params=pltpu.CompilerParams(use_tc_tiling_on_sc=False, needs_layout_passes=False),
        scratch_types=[
            pltpu.VMEM_SHARED((N, C), jnp.float32),
            pltpu.VMEM((CNT0, KC), jnp.int32),
            pltpu.VMEM((CNT0, KC), jnp.int32),
            pltpu.VMEM((CNT0, KC), jnp.float32),
            pltpu.VMEM((NB, KC, C), jnp.bfloat16 if BF else jnp.float32),
            pltpu.VMEM((NB, KC, C), jnp.float32),
        ] + [pltpu.SemaphoreType.DMA] * (2 * NB),
    )


def _make_sc_deg():
    def body(src_hbm, dst_hbm, w_hbm, z_hbm, out_hbm,
             acc, srcm, dstm, wm, rows, sem0, sem1):
        ssems = (sem0, sem1)
        cid = lax.axis_index("c")
        sid = lax.axis_index("s")
        gwid = cid * NS + sid
        _rowwise_copy(lambda sl: pltpu.sync_copy(z_hbm.at[sl], acc.at[sl]), sid)
        _stage_edges(src_hbm, dst_hbm, w_hbm, srcm, dstm, wm, gwid)
        plsc.subcore_barrier()

        def visit(cur, b):
            @pl.when(cur >= 2)
            def _():
                pltpu.make_async_copy(
                    rows.at[b], acc.at[srcm.at[cur - 2]], ssems[b]).wait()

            def gbody(g, cc):
                w16 = wm[cur, pl.ds(g * L, L)]
                for j in range(L):
                    e = g * L + j
                    rows[b, e, pl.ds(0, L)] = jnp.full((L,), w16[j],
                                                       jnp.float32)
                return cc

            lax.fori_loop(0, K // L, gbody, 0)
            pltpu.async_copy(rows.at[b], acc.at[srcm.at[cur]], ssems[b],
                             add=True)

        def outer(i, carry):
            for b in range(2):
                visit(i * 2 + b, b)
            return carry

        lax.fori_loop(0, NCHUNK // 2, outer, 0)
        for tail in range(NCHUNK - 2, NCHUNK):
            pltpu.make_async_copy(
                rows.at[tail % 2], acc.at[srcm.at[tail]],
                ssems[tail % 2]).wait()
        plsc.subcore_barrier()
        _rowwise_copy(
            lambda sl: pltpu.sync_copy(acc.at[sl], out_hbm.at[cid, sl]), sid)

    return pl.kernel(
        body,
        out_type=jax.ShapeDtypeStruct((NC, N, L), jnp.float32),
        mesh=_sc_mesh(),
        compiler_params=pltpu.CompilerParams(use_tc_tiling_on_sc=False, needs_layout_passes=False),
        scratch_types=[
            pltpu.VMEM_SHARED((N, L), jnp.float32),
            pltpu.VMEM((NCHUNK, K), jnp.int32),
            pltpu.VMEM((NCHUNK, K), jnp.int32),
            pltpu.VMEM((NCHUNK, K), jnp.float32),
            pltpu.VMEM((2, K, L), jnp.float32),
            pltpu.SemaphoreType.DMA,
            pltpu.SemaphoreType.DMA,
        ],
    )


def _tc_pre(degp, x):
    def body(degp_ref, x_ref, dis_ref, a_ref):
        d = degp_ref[0] + degp_ref[1]
        pos = d > 0.0
        safe = jnp.where(pos, d, 1.0)
        dis = jnp.where(pos, lax.rsqrt(safe), 0.0)
        dis_ref[...] = dis
        a_ref[...] = (x_ref[...] * dis[:, 0:1]).astype(jnp.bfloat16)

    return pl.pallas_call(
        body,
        grid=(N // BN,),
        in_specs=[
            pl.BlockSpec((2, BN, L), lambda i: (0, i, 0)),
            pl.BlockSpec((BN, 128), lambda i: (i, 0)),
        ],
        out_specs=[
            pl.BlockSpec((BN, L), lambda i: (i, 0)),
            pl.BlockSpec((BN, 128), lambda i: (i, 0)),
        ],
        out_shape=[
            jax.ShapeDtypeStruct((N, L), jnp.float32),
            jax.ShapeDtypeStruct((N, 128), jnp.bfloat16),
        ],
    )(degp, x)


def _tc_mid(r1p, h, dis, W4s, Ci, Co, tp_dtype):
    def body(r_ref, h_ref, dis_ref, w_ref, op_ref, tp_ref):
        d = dis_ref[:, 0:1]
        y1 = -d * (r_ref[0] + r_ref[1])
        v0 = w_ref[0] - w_ref[1]
        op = jnp.dot(h_ref[...], v0, preferred_element_type=jnp.float32)
        op = op + jnp.dot(y1, w_ref[2], preferred_element_type=jnp.float32)
        op_ref[...] = op
        tp = d * jnp.dot(y1, w_ref[3], preferred_element_type=jnp.float32)
        tp_ref[...] = tp.astype(tp_dtype)

    return pl.pallas_call(
        body,
        grid=(N // BN,),
        in_specs=[
            pl.BlockSpec((2, BN, Ci), lambda i: (0, i, 0)),
            pl.BlockSpec((BN, Ci), lambda i: (i, 0)),
            pl.BlockSpec((BN, L), lambda i: (i, 0)),
            pl.BlockSpec((4, Ci, Co), lambda i: (0, 0, 0)),
        ],
        out_specs=[
            pl.BlockSpec((BN, Co), lambda i: (i, 0)),
            pl.BlockSpec((BN, Co), lambda i: (i, 0)),
        ],
        out_shape=[
            jax.ShapeDtypeStruct((N, Co), jnp.float32),
            jax.ShapeDtypeStruct((N, Co), tp_dtype),
        ],
    )(r1p, h, dis, W4s)


def _tc_post(r2p, op, dis, b8, Co, a_dtype):
    def body(r_ref, op_ref, dis_ref, b_ref, h_ref, a_ref):
        d = dis_ref[:, 0:1]
        o = op_ref[...] + b_ref[0:1, :] - 2.0 * d * (r_ref[0] + r_ref[1])
        h = jnp.maximum(o, 0.0)
        h_ref[...] = h
        a_ref[...] = (d * h).astype(a_dtype)

    return pl.pallas_call(
        body,
        grid=(N // BN,),
        in_specs=[
            pl.BlockSpec((2, BN, Co), lambda i: (0, i, 0)),
            pl.BlockSpec((BN, Co), lambda i: (i, 0)),
            pl.BlockSpec((BN, L), lambda i: (i, 0)),
            pl.BlockSpec((8, Co), lambda i: (0, 0)),
        ],
        out_specs=[
            pl.BlockSpec((BN, Co), lambda i: (i, 0)),
            pl.BlockSpec((BN, Co), lambda i: (i, 0)),
        ],
        out_shape=[
            jax.ShapeDtypeStruct((N, Co), jnp.float32),
            jax.ShapeDtypeStruct((N, Co), a_dtype),
        ],
    )(r2p, op, dis, b8)


def _tc_final(r2p, op, dis, b8):
    def body(r_ref, op_ref, dis_ref, b_ref, out_ref):
        d = dis_ref[:, 0:1]
        o = op_ref[...] + b_ref[0:1, :] - 2.0 * d * (r_ref[0] + r_ref[1])
        z = o[:, :10]
        m = jnp.max(z, axis=1, keepdims=True)
        zs = z - m
        lse = jnp.log(jnp.sum(jnp.exp(zs), axis=1, keepdims=True))
        out_ref[...] = zs - lse

    return pl.pallas_call(
        body,
        grid=(N // BN,),
        in_specs=[
            pl.BlockSpec((2, BN, 16), lambda i: (0, i, 0)),
            pl.BlockSpec((BN, 16), lambda i: (i, 0)),
            pl.BlockSpec((BN, L), lambda i: (i, 0)),
            pl.BlockSpec((8, 16), lambda i: (0, 0)),
        ],
        out_specs=pl.BlockSpec((BN, 10), lambda i: (i, 0)),
        out_shape=jax.ShapeDtypeStruct((N, 10), jnp.float32),
    )(r2p, op, dis, b8)


_sc_pass = {C: _make_sc_edge_pass(C) for C in (128, 64, 32, 16)}
_sc_deg = _make_sc_deg()


def kernel(x, edge_index, edge_weight, W1, b1, W2, b2, W3, b3, W4, b4):
    zpad_i = jnp.zeros((E_PAD,), jnp.int32)
    srcf = jnp.concatenate([edge_index[0], zpad_i])
    dstf = jnp.concatenate([edge_index[1], zpad_i])
    wf = jnp.concatenate([edge_weight, jnp.zeros((E_PAD,), jnp.float32)])

    def eshape(C):
        kc = 16 if C == 128 else K
        return ((NW * EPT) // kc, kc)

    edges = {C: (srcf.reshape(eshape(C)), dstf.reshape(eshape(C)),
                 wf.reshape(eshape(C)))
             for C in (128, 64, 32, 16)}
    src3, dst3, w3 = edges[16]
    zeros = {C: jnp.zeros((N, C), jnp.float32) for C in (128, 64, 32, 16)}

    W4p = jnp.zeros((3, 16, 16), jnp.float32).at[:, :, :10].set(W4)
    b4p = jnp.zeros((16,), jnp.float32).at[:10].set(b4)

    degp = _sc_deg(src3, dst3, w3, zeros[16])
    dis, a = _tc_pre(degp, x)

    layers = [
        (W1, b1, 128, 64),
        (W2, b2, 64, 32),
        (W3, b3, 32, 16),
        (W4p, b4p, 16, 16),
    ]
    perm = {c: np.concatenate([np.arange(0, c, 2), np.arange(1, c, 2)])
            for c in (128, 64, 32)}
    iperm = {c: np.argsort(perm[c]) for c in (128, 64, 32)}

    h = x
    for li, (W, b, Ci, Co) in enumerate(layers):
        b8 = jnp.broadcast_to(b, (8, Co))
        w1p = W[1][perm[Ci], :] if Ci in perm else W[1]
        w2p = W[2][perm[Ci], :] if Ci in perm else W[2]
        w2pp = w2p[:, iperm[Co]] if Co in perm else w2p
        w4s = jnp.stack([W[0], W[2], w1p, w2pp])
        tp_dtype = jnp.bfloat16 if Co >= 32 else jnp.float32
        r1p = _sc_pass[Ci](a, *edges[Ci], zeros[Ci])
        op, tp = _tc_mid(r1p, h, dis, w4s, Ci, Co, tp_dtype)
        r2p = _sc_pass[Co](tp, *edges[Co], zeros[Co])
        if li < 3:
            a_dtype = jnp.bfloat16 if Co >= 32 else jnp.float32
            h, a = _tc_post(r2p, op, dis, b8, Co, a_dtype)
        else:
            return _tc_final(r2p, op, dis, b8)


# speedup vs baseline: 1.2893x; 1.0696x over previous
import functools

import jax
import jax.numpy as jnp
import numpy as np
from jax import lax
from jax.experimental import pallas as pl
from jax.experimental.pallas import tpu as pltpu
from jax.experimental.pallas import tpu_sc as plsc

N = 10000
E = 320000
NC = 2
NS = 16
NW = NC * NS
K = 128
NCHUNK = 80
EPT = NCHUNK * K
E_PAD = NW * EPT - E
NBUF = 2
RPT = 624
TAIL = N - NS * RPT
L = 16
BN = 1000


def _sc_mesh():
    return plsc.VectorSubcoreMesh(core_axis_name="c", subcore_axis_name="s")


def _stage_edges(src_hbm, dst_hbm, w_hbm, srcm, dstm, wm, gwid):
    base = gwid * NCHUNK
    pltpu.sync_copy(src_hbm.at[pl.ds(base, NCHUNK)], srcm)
    pltpu.sync_copy(dst_hbm.at[pl.ds(base, NCHUNK)], dstm)
    pltpu.sync_copy(w_hbm.at[pl.ds(base, NCHUNK)], wm)

    def mask_chunk(i, carry):
        for g in range(K // L):
            sl = pl.ds(g * L, L)
            s16 = srcm[i, sl]
            d16 = dstm[i, sl]
            w16 = wm[i, sl]
            wm[i, sl] = jnp.where(s16 == d16, jnp.zeros((L,), jnp.float32), w16)
        return carry

    lax.fori_loop(0, NCHUNK, mask_chunk, 0)


def _rowwise_copy(copy_fn, sid):
    copy_fn(pl.ds(sid * RPT, RPT))

    @pl.when(sid == NS - 1)
    def _():
        copy_fn(pl.ds(NS * RPT, TAIL))


def _make_sc_edge_pass(C):
    KC = 16 if C == 128 else K
    TOTCH = (NW * EPT) // KC
    NB, LEAD = 4, 2
    BF = C >= 32
    if C == 128:
        CNT0, CNT1 = 744, 536
    elif C in (64, 32):
        CNT0, CNT1 = 92, 68
    else:
        CNT0, CNT1 = 112, 48

    def body(y_hbm, src_hbm, dst_hbm, w_hbm, z_hbm, out_hbm,
             acc, srcm, dstm, wm, rows, srow, *sems):
        gsems = sems[:NB]
        ssems = sems[NB:]
        cid = lax.axis_index("c")
        sid = lax.axis_index("s")
        _rowwise_copy(lambda sl: pltpu.sync_copy(z_hbm.at[sl], acc.at[sl]), sid)
        start = jnp.where(cid == 0, sid * CNT0, NS * CNT0 + sid * CNT1)
        nch = jnp.where(cid == 0, CNT0, CNT1)

        @pl.when(cid == 0)
        def _():
            pltpu.sync_copy(src_hbm.at[pl.ds(start, CNT0)], srcm)
            pltpu.sync_copy(dst_hbm.at[pl.ds(start, CNT0)], dstm)
            pltpu.sync_copy(w_hbm.at[pl.ds(start, CNT0)], wm)

        @pl.when(cid == 1)
        def _():
            csl = pl.ds(0, CNT1)
            pltpu.sync_copy(src_hbm.at[pl.ds(start, CNT1)], srcm.at[csl])
            pltpu.sync_copy(dst_hbm.at[pl.ds(start, CNT1)], dstm.at[csl])
            pltpu.sync_copy(w_hbm.at[pl.ds(start, CNT1)], wm.at[csl])

        def mask_chunk(i, carry):
            for g in range(KC // L):
                sl = pl.ds(g * L, L)
                wm[i, sl] = jnp.where(srcm[i, sl] == dstm[i, sl],
                                      jnp.zeros((L,), jnp.float32), wm[i, sl])
            return carry

        lax.fori_loop(0, nch, mask_chunk, 0)
        plsc.subcore_barrier()

        for c0 in range(LEAD):
            pltpu.async_copy(y_hbm.at[srcm.at[c0]], rows.at[c0], gsems[c0])

        def visit(cur, b):
            pltpu.make_async_copy(
                y_hbm.at[srcm.at[cur]], rows.at[b], gsems[b]).wait()
            prev = cur - NB

            @pl.when(prev >= 0)
            def _():
                pltpu.make_async_copy(
                    srow.at[b], acc.at[dstm.at[prev]], ssems[b]).wait()

            def gbody(g, cc):
                w16 = wm[cur, pl.ds(g * L, L)]
                for j in range(L):
                    we = w16[j]
                    e = g * L + j
                    if BF:
                        for ci in range(C // (2 * L)):
                            v = rows[b, e, pl.ds(ci * 2 * L, 2 * L)]
                            lo, hi = plsc.unpack(
                                v, format=plsc.PackFormat.INTERLEAVED)
                            srow[b, e, pl.ds(ci * L, L)] = lo * we
                            srow[b, e, pl.ds(C // 2 + ci * L, L)] = hi * we
                    else:
                        for ci in range(C // L):
                            csl = pl.ds(ci * L, L)
                            srow[b, e, csl] = rows[b, e, csl] * we
                return cc

            lax.fori_loop(0, KC // L, gbody, 0)
            pltpu.async_copy(srow.at[b], acc.at[dstm.at[cur]], ssems[b],
                             add=True)
            nxt = cur + LEAD
            bj = (b + LEAD) % NB

            @pl.when(nxt < nch)
            def _():
                pltpu.async_copy(y_hbm.at[srcm.at[nxt]], rows.at[bj],
                                 gsems[bj])

        def outer(i, carry):
            for b in range(NB):
                visit(i * NB + b, b)
            return carry

        lax.fori_loop(0, nch // NB, outer, 0)
        for t in range(NB):
            pltpu.make_async_copy(
                srow.at[t], acc.at[dstm.at[nch - NB + t]],
                ssems[t]).wait()
        plsc.subcore_barrier()
        _rowwise_copy(
            lambda sl: pltpu.sync_copy(acc.at[sl], out_hbm.at[cid, sl]), sid)

    return pl.kernel(
        body,
        out_type=jax.ShapeDtypeStruct((NC, N, C), jnp.float32),
        mesh=_sc_mesh(),
        compiler_params=pltpu.CompilerParams(use_tc_tiling_on_sc=False, needs_layout_passes=False),
        scratch_types=[
            pltpu.VMEM_SHARED((N, C), jnp.float32),
            pltpu.VMEM((CNT0, KC), jnp.int32),
            pltpu.VMEM((CNT0, KC), jnp.int32),
            pltpu.VMEM((CNT0, KC), jnp.float32),
            pltpu.VMEM((NB, KC, C), jnp.bfloat16 if BF else jnp.float32),
            pltpu.VMEM((NB, KC, C), jnp.float32),
        ] + [pltpu.SemaphoreType.DMA] * (2 * NB),
    )


def _make_sc_deg():
    def body(src_hbm, dst_hbm, w_hbm, z_hbm, out_hbm,
             acc, srcm, dstm, wm, rows, sem0, sem1):
        ssems = (sem0, sem1)
        cid = lax.axis_index("c")
        sid = lax.axis_index("s")
        gwid = cid * NS + sid
        _rowwise_copy(lambda sl: pltpu.sync_copy(z_hbm.at[sl], acc.at[sl]), sid)
        _stage_edges(src_hbm, dst_hbm, w_hbm, srcm, dstm, wm, gwid)
        plsc.subcore_barrier()

        def visit(cur, b):
            @pl.when(cur >= 2)
            def _():
                pltpu.make_async_copy(
                    rows.at[b], acc.at[srcm.at[cur - 2]], ssems[b]).wait()

            def gbody(g, cc):
                w16 = wm[cur, pl.ds(g * L, L)]
                for j in range(L):
                    e = g * L + j
                    rows[b, e, pl.ds(0, L)] = jnp.full((L,), w16[j],
                                                       jnp.float32)
                return cc

            lax.fori_loop(0, K // L, gbody, 0)
            pltpu.async_copy(rows.at[b], acc.at[srcm.at[cur]], ssems[b],
                             add=True)

        def outer(i, carry):
            for b in range(2):
                visit(i * 2 + b, b)
            return carry

        lax.fori_loop(0, NCHUNK // 2, outer, 0)
        for tail in range(NCHUNK - 2, NCHUNK):
            pltpu.make_async_copy(
                rows.at[tail % 2], acc.at[srcm.at[tail]],
                ssems[tail % 2]).wait()
        plsc.subcore_barrier()
        _rowwise_copy(
            lambda sl: pltpu.sync_copy(acc.at[sl], out_hbm.at[cid, sl]), sid)

    return pl.kernel(
        body,
        out_type=jax.ShapeDtypeStruct((NC, N, L), jnp.float32),
        mesh=_sc_mesh(),
        compiler_params=pltpu.CompilerParams(use_tc_tiling_on_sc=False, needs_layout_passes=False),
        scratch_types=[
            pltpu.VMEM_SHARED((N, L), jnp.float32),
            pltpu.VMEM((NCHUNK, K), jnp.int32),
            pltpu.VMEM((NCHUNK, K), jnp.int32),
            pltpu.VMEM((NCHUNK, K), jnp.float32),
            pltpu.VMEM((2, K, L), jnp.float32),
            pltpu.SemaphoreType.DMA,
            pltpu.SemaphoreType.DMA,
        ],
    )


def _tc_pre(degp, x):
    def body(degp_ref, x_ref, dis_ref, a_ref):
        d = degp_ref[0] + degp_ref[1]
        pos = d > 0.0
        safe = jnp.where(pos, d, 1.0)
        dis = jnp.where(pos, lax.rsqrt(safe), 0.0)
        dis_ref[...] = dis
        a_ref[...] = (x_ref[...] * dis[:, 0:1]).astype(jnp.bfloat16)

    return pl.pallas_call(
        body,
        grid=(N // BN,),
        in_specs=[
            pl.BlockSpec((2, BN, L), lambda i: (0, i, 0)),
            pl.BlockSpec((BN, 128), lambda i: (i, 0)),
        ],
        out_specs=[
            pl.BlockSpec((BN, L), lambda i: (i, 0)),
            pl.BlockSpec((BN, 128), lambda i: (i, 0)),
        ],
        out_shape=[
            jax.ShapeDtypeStruct((N, L), jnp.float32),
            jax.ShapeDtypeStruct((N, 128), jnp.bfloat16),
        ],
    )(degp, x)


def _tc_mid(r1p, h, dis, W4s, Ci, Co, tp_dtype):
    def body(r_ref, h_ref, dis_ref, w_ref, op_ref, tp_ref):
        d = dis_ref[:, 0:1]
        y1 = -d * (r_ref[0] + r_ref[1])
        v0 = w_ref[0] - w_ref[1]
        op = jnp.dot(h_ref[...], v0, preferred_element_type=jnp.float32)
        op = op + jnp.dot(y1, w_ref[2], preferred_element_type=jnp.float32)
        op_ref[...] = op
        tp = d * jnp.dot(y1, w_ref[3], preferred_element_type=jnp.float32)
        tp_ref[...] = tp.astype(tp_dtype)

    return pl.pallas_call(
        body,
        grid=(N // BN,),
        in_specs=[
            pl.BlockSpec((2, BN, Ci), lambda i: (0, i, 0)),
            pl.BlockSpec((BN, Ci), lambda i: (i, 0)),
            pl.BlockSpec((BN, L), lambda i: (i, 0)),
            pl.BlockSpec((4, Ci, Co), lambda i: (0, 0, 0)),
        ],
        out_specs=[
            pl.BlockSpec((BN, Co), lambda i: (i, 0)),
            pl.BlockSpec((BN, Co), lambda i: (i, 0)),
        ],
        out_shape=[
            jax.ShapeDtypeStruct((N, Co), jnp.float32),
            jax.ShapeDtypeStruct((N, Co), tp_dtype),
        ],
    )(r1p, h, dis, W4s)


def _tc_post(r2p, op, dis, b8, Co, a_dtype):
    def body(r_ref, op_ref, dis_ref, b_ref, h_ref, a_ref):
        d = dis_ref[:, 0:1]
        o = op_ref[...] + b_ref[0:1, :] - 2.0 * d * (r_ref[0] + r_ref[1])
        h = jnp.maximum(o, 0.0)
        h_ref[...] = h
        a_ref[...] = (d * h).astype(a_dtype)

    return pl.pallas_call(
        body,
        grid=(N // BN,),
        in_specs=[
            pl.BlockSpec((2, BN, Co), lambda i: (0, i, 0)),
            pl.BlockSpec((BN, Co), lambda i: (i, 0)),
            pl.BlockSpec((BN, L), lambda i: (i, 0)),
            pl.BlockSpec((8, Co), lambda i: (0, 0)),
        ],
        out_specs=[
            pl.BlockSpec((BN, Co), lambda i: (i, 0)),
            pl.BlockSpec((BN, Co), lambda i: (i, 0)),
        ],
        out_shape=[
            jax.ShapeDtypeStruct((N, Co), jnp.float32),
            jax.ShapeDtypeStruct((N, Co), a_dtype),
        ],
    )(r2p, op, dis, b8)


def _tc_final(r2p, op, dis, b8):
    def body(r_ref, op_ref, dis_ref, b_ref, out_ref):
        d = dis_ref[:, 0:1]
        o = op_ref[...] + b_ref[0:1, :] - 2.0 * d * (r_ref[0] + r_ref[1])
        z = o[:, :10]
        m = jnp.max(z, axis=1, keepdims=True)
        zs = z - m
        lse = jnp.log(jnp.sum(jnp.exp(zs), axis=1, keepdims=True))
        out_ref[...] = zs - lse

    return pl.pallas_call(
        body,
        grid=(N // BN,),
        in_specs=[
            pl.BlockSpec((2, BN, 16), lambda i: (0, i, 0)),
            pl.BlockSpec((BN, 16), lambda i: (i, 0)),
            pl.BlockSpec((BN, L), lambda i: (i, 0)),
            pl.BlockSpec((8, 16), lambda i: (0, 0)),
        ],
        out_specs=pl.BlockSpec((BN, 10), lambda i: (i, 0)),
        out_shape=jax.ShapeDtypeStruct((N, 10), jnp.float32),
    )(r2p, op, dis, b8)


_sc_pass = {C: _make_sc_edge_pass(C) for C in (128, 64, 32, 16)}
_sc_deg = _make_sc_deg()


def kernel(x, edge_index, edge_weight, W1, b1, W2, b2, W3, b3, W4, b4):
    zpad_i = jnp.zeros((E_PAD,), jnp.int32)
    srcf = jnp.concatenate([edge_index[0], zpad_i])
    dstf = jnp.concatenate([edge_index[1], zpad_i])
    wf = jnp.concatenate([edge_weight, jnp.zeros((E_PAD,), jnp.float32)])

    def eshape(C):
        kc = 16 if C == 128 else K
        return ((NW * EPT) // kc, kc)

    edges = {C: (srcf.reshape(eshape(C)), dstf.reshape(eshape(C)),
                 wf.reshape(eshape(C)))
             for C in (128, 64, 32, 16)}
    src3, dst3, w3 = edges[16]
    zeros = {C: jnp.zeros((N, C), jnp.float32) for C in (128, 64, 32, 16)}

    W4p = jnp.zeros((3, 16, 16), jnp.float32).at[:, :, :10].set(W4)
    b4p = jnp.zeros((16,), jnp.float32).at[:10].set(b4)

    degp = _sc_deg(src3, dst3, w3, zeros[16])
    dis, a = _tc_pre(degp, x)

    layers = [
        (W1, b1, 128, 64),
        (W2, b2, 64, 32),
        (W3, b3, 32, 16),
        (W4p, b4p, 16, 16),
    ]
    perm = {c: np.concatenate([np.arange(0, c, 2), np.arange(1, c, 2)])
            for c in (128, 64, 32)}
    iperm = {c: np.argsort(perm[c]) for c in (128, 64, 32)}

    h = x
    for li, (W, b, Ci, Co) in enumerate(layers):
        b8 = jnp.broadcast_to(b, (8, Co))
        w1p = W[1][perm[Ci], :] if Ci in perm else W[1]
        w2p = W[2][perm[Ci], :] if Ci in perm else W[2]
        w2pp = w2p[:, iperm[Co]] if Co in perm else w2p
        w4s = jnp.stack([W[0], W[2], w1p, w2pp])
        tp_dtype = jnp.bfloat16 if Co >= 32 else jnp.float32
        r1p = _sc_pass[Ci](a, *edges[Ci], zeros[Ci])
        op, tp = _tc_mid(r1p, h, dis, w4s, Ci, Co, tp_dtype)
        r2p = _sc_pass[Co](tp, *edges[Co], zeros[Co])
        if li < 3:
            a_dtype = jnp.bfloat16 if Co >= 32 else jnp.float32
            h, a = _tc_post(r2p, op, dis, b8, Co, a_dtype)
        else:
            return _tc_final(r2p, op, dis, b8)


# speedup vs baseline: 1.2943x; 1.0038x over previous
import functools

import jax
import jax.numpy as jnp
import numpy as np
from jax import lax
from jax.experimental import pallas as pl
from jax.experimental.pallas import tpu as pltpu
from jax.experimental.pallas import tpu_sc as plsc

N = 10000
E = 320000
NC = 2
NS = 16
NW = NC * NS
K = 128
NCHUNK = 80
EPT = NCHUNK * K
E_PAD = NW * EPT - E
NBUF = 2
RPT = 624
TAIL = N - NS * RPT
L = 16
BN = 1000


def _sc_mesh():
    return plsc.VectorSubcoreMesh(core_axis_name="c", subcore_axis_name="s")


def _stage_edges(src_hbm, dst_hbm, w_hbm, srcm, dstm, wm, gwid):
    base = gwid * NCHUNK
    pltpu.sync_copy(src_hbm.at[pl.ds(base, NCHUNK)], srcm)
    pltpu.sync_copy(dst_hbm.at[pl.ds(base, NCHUNK)], dstm)
    pltpu.sync_copy(w_hbm.at[pl.ds(base, NCHUNK)], wm)

    def mask_chunk(i, carry):
        for g in range(K // L):
            sl = pl.ds(g * L, L)
            s16 = srcm[i, sl]
            d16 = dstm[i, sl]
            w16 = wm[i, sl]
            wm[i, sl] = jnp.where(s16 == d16, jnp.zeros((L,), jnp.float32), w16)
        return carry

    lax.fori_loop(0, NCHUNK, mask_chunk, 0)


def _rowwise_copy(copy_fn, sid):
    copy_fn(pl.ds(sid * RPT, RPT))

    @pl.when(sid == NS - 1)
    def _():
        copy_fn(pl.ds(NS * RPT, TAIL))


def _make_sc_edge_pass(C):
    KC = 16 if C == 128 else K
    TOTCH = (NW * EPT) // KC
    NB, LEAD = 4, 2
    BF = C >= 32
    if C == 128:
        CNT0, CNT1 = 736, 544
    elif C in (64, 32):
        CNT0, CNT1 = 88, 72
    else:
        CNT0, CNT1 = 112, 48

    def body(y_hbm, src_hbm, dst_hbm, w_hbm, z_hbm, out_hbm,
             acc, srcm, dstm, wm, rows, srow, *sems):
        gsems = sems[:NB]
        ssems = sems[NB:]
        cid = lax.axis_index("c")
        sid = lax.axis_index("s")
        _rowwise_copy(lambda sl: pltpu.sync_copy(z_hbm.at[sl], acc.at[sl]), sid)
        start = jnp.where(cid == 0, sid * CNT0, NS * CNT0 + sid * CNT1)
        nch = jnp.where(cid == 0, CNT0, CNT1)

        @pl.when(cid == 0)
        def _():
            pltpu.sync_copy(src_hbm.at[pl.ds(start, CNT0)], srcm)
            pltpu.sync_copy(dst_hbm.at[pl.ds(start, CNT0)], dstm)
            pltpu.sync_copy(w_hbm.at[pl.ds(start, CNT0)], wm)

        @pl.when(cid == 1)
        def _():
            csl = pl.ds(0, CNT1)
            pltpu.sync_copy(src_hbm.at[pl.ds(start, CNT1)], srcm.at[csl])
            pltpu.sync_copy(dst_hbm.at[pl.ds(start, CNT1)], dstm.at[csl])
            pltpu.sync_copy(w_hbm.at[pl.ds(start, CNT1)], wm.at[csl])

        def mask_chunk(i, carry):
            for g in range(KC // L):
                sl = pl.ds(g * L, L)
                wm[i, sl] = jnp.where(srcm[i, sl] == dstm[i, sl],
                                      jnp.zeros((L,), jnp.float32), wm[i, sl])
            return carry

        lax.fori_loop(0, nch, mask_chunk, 0)
        plsc.subcore_barrier()

        for c0 in range(LEAD):
            pltpu.async_copy(y_hbm.at[srcm.at[c0]], rows.at[c0], gsems[c0])

        def visit(cur, b):
            pltpu.make_async_copy(
                y_hbm.at[srcm.at[cur]], rows.at[b], gsems[b]).wait()
            prev = cur - NB

            @pl.when(prev >= 0)
            def _():
                pltpu.make_async_copy(
                    srow.at[b], acc.at[dstm.at[prev]], ssems[b]).wait()

            def gbody(g, cc):
                w16 = wm[cur, pl.ds(g * L, L)]
                for j in range(L):
                    we = w16[j]
                    e = g * L + j
                    if BF:
                        for ci in range(C // (2 * L)):
                            v = rows[b, e, pl.ds(ci * 2 * L, 2 * L)]
                            lo, hi = plsc.unpack(
                                v, format=plsc.PackFormat.INTERLEAVED)
                            srow[b, e, pl.ds(ci * L, L)] = lo * we
                            srow[b, e, pl.ds(C // 2 + ci * L, L)] = hi * we
                    else:
                        for ci in range(C // L):
                            csl = pl.ds(ci * L, L)
                            srow[b, e, csl] = rows[b, e, csl] * we
                return cc

            lax.fori_loop(0, KC // L, gbody, 0)
            pltpu.async_copy(srow.at[b], acc.at[dstm.at[cur]], ssems[b],
                             add=True)
            nxt = cur + LEAD
            bj = (b + LEAD) % NB

            @pl.when(nxt < nch)
            def _():
                pltpu.async_copy(y_hbm.at[srcm.at[nxt]], rows.at[bj],
                                 gsems[bj])

        def outer(i, carry):
            for b in range(NB):
                visit(i * NB + b, b)
            return carry

        lax.fori_loop(0, nch // NB, outer, 0)
        for t in range(NB):
            pltpu.make_async_copy(
                srow.at[t], acc.at[dstm.at[nch - NB + t]],
                ssems[t]).wait()
        plsc.subcore_barrier()
        _rowwise_copy(
            lambda sl: pltpu.sync_copy(acc.at[sl], out_hbm.at[cid, sl]), sid)

    return pl.kernel(
        body,
        out_type=jax.ShapeDtypeStruct((NC, N, C), jnp.float32),
        mesh=_sc_mesh(),
        compiler_params=pltpu.CompilerParams(use_tc_tiling_on_sc=False, needs_layout_passes=False),
        scratch_types=[
            pltpu.VMEM_SHARED((N, C), jnp.float32),
            pltpu.VMEM((CNT0, KC), jnp.int32),
            pltpu.VMEM((CNT0, KC), jnp.int32),
            pltpu.VMEM((CNT0, KC), jnp.float32),
            pltpu.VMEM((NB, KC, C), jnp.bfloat16 if BF else jnp.float32),
            pltpu.VMEM((NB, KC, C), jnp.float32),
        ] + [pltpu.SemaphoreType.DMA] * (2 * NB),
    )


def _make_sc_deg():
    def body(src_hbm, dst_hbm, w_hbm, z_hbm, out_hbm,
             acc, srcm, dstm, wm, rows, sem0, sem1):
        ssems = (sem0, sem1)
        cid = lax.axis_index("c")
        sid = lax.axis_index("s")
        gwid = cid * NS + sid
        _rowwise_copy(lambda sl: pltpu.sync_copy(z_hbm.at[sl], acc.at[sl]), sid)
        _stage_edges(src_hbm, dst_hbm, w_hbm, srcm, dstm, wm, gwid)
        plsc.subcore_barrier()

        def visit(cur, b):
            @pl.when(cur >= 2)
            def _():
                pltpu.make_async_copy(
                    rows.at[b], acc.at[srcm.at[cur - 2]], ssems[b]).wait()

            def gbody(g, cc):
                w16 = wm[cur, pl.ds(g * L, L)]
                for j in range(L):
                    e = g * L + j
                    rows[b, e, pl.ds(0, L)] = jnp.full((L,), w16[j],
                                                       jnp.float32)
                return cc

            lax.fori_loop(0, K // L, gbody, 0)
            pltpu.async_copy(rows.at[b], acc.at[srcm.at[cur]], ssems[b],
                             add=True)

        def outer(i, carry):
            for b in range(2):
                visit(i * 2 + b, b)
            return carry

        lax.fori_loop(0, NCHUNK // 2, outer, 0)
        for tail in range(NCHUNK - 2, NCHUNK):
            pltpu.make_async_copy(
                rows.at[tail % 2], acc.at[srcm.at[tail]],
                ssems[tail % 2]).wait()
        plsc.subcore_barrier()
        _rowwise_copy(
            lambda sl: pltpu.sync_copy(acc.at[sl], out_hbm.at[cid, sl]), sid)

    return pl.kernel(
        body,
        out_type=jax.ShapeDtypeStruct((NC, N, L), jnp.float32),
        mesh=_sc_mesh(),
        compiler_params=pltpu.CompilerParams(use_tc_tiling_on_sc=False, needs_layout_passes=False),
        scratch_types=[
            pltpu.VMEM_SHARED((N, L), jnp.float32),
            pltpu.VMEM((NCHUNK, K), jnp.int32),
            pltpu.VMEM((NCHUNK, K), jnp.int32),
            pltpu.VMEM((NCHUNK, K), jnp.float32),
            pltpu.VMEM((2, K, L), jnp.float32),
            pltpu.SemaphoreType.DMA,
            pltpu.SemaphoreType.DMA,
        ],
    )


def _tc_pre(degp, x):
    def body(degp_ref, x_ref, dis_ref, a_ref):
        d = degp_ref[0] + degp_ref[1]
        pos = d > 0.0
        safe = jnp.where(pos, d, 1.0)
        dis = jnp.where(pos, lax.rsqrt(safe), 0.0)
        dis_ref[...] = dis
        a_ref[...] = (x_ref[...] * dis[:, 0:1]).astype(jnp.bfloat16)

    return pl.pallas_call(
        body,
        grid=(N // BN,),
        in_specs=[
            pl.BlockSpec((2, BN, L), lambda i: (0, i, 0)),
            pl.BlockSpec((BN, 128), lambda i: (i, 0)),
        ],
        out_specs=[
            pl.BlockSpec((BN, L), lambda i: (i, 0)),
            pl.BlockSpec((BN, 128), lambda i: (i, 0)),
        ],
        out_shape=[
            jax.ShapeDtypeStruct((N, L), jnp.float32),
            jax.ShapeDtypeStruct((N, 128), jnp.bfloat16),
        ],
    )(degp, x)


def _tc_mid(r1p, h, dis, W4s, Ci, Co, tp_dtype):
    def body(r_ref, h_ref, dis_ref, w_ref, op_ref, tp_ref):
        d = dis_ref[:, 0:1]
        y1 = -d * (r_ref[0] + r_ref[1])
        v0 = w_ref[0] - w_ref[1]
        op = jnp.dot(h_ref[...], v0, preferred_element_type=jnp.float32)
        op = op + jnp.dot(y1, w_ref[2], preferred_element_type=jnp.float32)
        op_ref[...] = op
        tp = d * jnp.dot(y1, w_ref[3], preferred_element_type=jnp.float32)
        tp_ref[...] = tp.astype(tp_dtype)

    return pl.pallas_call(
        body,
        grid=(N // BN,),
        in_specs=[
            pl.BlockSpec((2, BN, Ci), lambda i: (0, i, 0)),
            pl.BlockSpec((BN, Ci), lambda i: (i, 0)),
            pl.BlockSpec((BN, L), lambda i: (i, 0)),
            pl.BlockSpec((4, Ci, Co), lambda i: (0, 0, 0)),
        ],
        out_specs=[
            pl.BlockSpec((BN, Co), lambda i: (i, 0)),
            pl.BlockSpec((BN, Co), lambda i: (i, 0)),
        ],
        out_shape=[
            jax.ShapeDtypeStruct((N, Co), jnp.float32),
            jax.ShapeDtypeStruct((N, Co), tp_dtype),
        ],
    )(r1p, h, dis, W4s)


def _tc_post(r2p, op, dis, b8, Co, a_dtype):
    def body(r_ref, op_ref, dis_ref, b_ref, h_ref, a_ref):
        d = dis_ref[:, 0:1]
        o = op_ref[...] + b_ref[0:1, :] - 2.0 * d * (r_ref[0] + r_ref[1])
        h = jnp.maximum(o, 0.0)
        h_ref[...] = h
        a_ref[...] = (d * h).astype(a_dtype)

    return pl.pallas_call(
        body,
        grid=(N // BN,),
        in_specs=[
            pl.BlockSpec((2, BN, Co), lambda i: (0, i, 0)),
            pl.BlockSpec((BN, Co), lambda i: (i, 0)),
            pl.BlockSpec((BN, L), lambda i: (i, 0)),
            pl.BlockSpec((8, Co), lambda i: (0, 0)),
        ],
        out_specs=[
            pl.BlockSpec((BN, Co), lambda i: (i, 0)),
            pl.BlockSpec((BN, Co), lambda i: (i, 0)),
        ],
        out_shape=[
            jax.ShapeDtypeStruct((N, Co), jnp.float32),
            jax.ShapeDtypeStruct((N, Co), a_dtype),
        ],
    )(r2p, op, dis, b8)


def _tc_final(r2p, op, dis, b8):
    def body(r_ref, op_ref, dis_ref, b_ref, out_ref):
        d = dis_ref[:, 0:1]
        o = op_ref[...] + b_ref[0:1, :] - 2.0 * d * (r_ref[0] + r_ref[1])
        z = o[:, :10]
        m = jnp.max(z, axis=1, keepdims=True)
        zs = z - m
        lse = jnp.log(jnp.sum(jnp.exp(zs), axis=1, keepdims=True))
        out_ref[...] = zs - lse

    return pl.pallas_call(
        body,
        grid=(N // BN,),
        in_specs=[
            pl.BlockSpec((2, BN, 16), lambda i: (0, i, 0)),
            pl.BlockSpec((BN, 16), lambda i: (i, 0)),
            pl.BlockSpec((BN, L), lambda i: (i, 0)),
            pl.BlockSpec((8, 16), lambda i: (0, 0)),
        ],
        out_specs=pl.BlockSpec((BN, 10), lambda i: (i, 0)),
        out_shape=jax.ShapeDtypeStruct((N, 10), jnp.float32),
    )(r2p, op, dis, b8)


_sc_pass = {C: _make_sc_edge_pass(C) for C in (128, 64, 32, 16)}
_sc_deg = _make_sc_deg()


def kernel(x, edge_index, edge_weight, W1, b1, W2, b2, W3, b3, W4, b4):
    zpad_i = jnp.zeros((E_PAD,), jnp.int32)
    srcf = jnp.concatenate([edge_index[0], zpad_i])
    dstf = jnp.concatenate([edge_index[1], zpad_i])
    wf = jnp.concatenate([edge_weight, jnp.zeros((E_PAD,), jnp.float32)])

    def eshape(C):
        kc = 16 if C == 128 else K
        return ((NW * EPT) // kc, kc)

    edges = {C: (srcf.reshape(eshape(C)), dstf.reshape(eshape(C)),
                 wf.reshape(eshape(C)))
             for C in (128, 64, 32, 16)}
    src3, dst3, w3 = edges[16]
    zeros = {C: jnp.zeros((N, C), jnp.float32) for C in (128, 64, 32, 16)}

    W4p = jnp.zeros((3, 16, 16), jnp.float32).at[:, :, :10].set(W4)
    b4p = jnp.zeros((16,), jnp.float32).at[:10].set(b4)

    degp = _sc_deg(src3, dst3, w3, zeros[16])
    dis, a = _tc_pre(degp, x)

    layers = [
        (W1, b1, 128, 64),
        (W2, b2, 64, 32),
        (W3, b3, 32, 16),
        (W4p, b4p, 16, 16),
    ]
    perm = {c: np.concatenate([np.arange(0, c, 2), np.arange(1, c, 2)])
            for c in (128, 64, 32)}
    iperm = {c: np.argsort(perm[c]) for c in (128, 64, 32)}

    h = x
    for li, (W, b, Ci, Co) in enumerate(layers):
        b8 = jnp.broadcast_to(b, (8, Co))
        w1p = W[1][perm[Ci], :] if Ci in perm else W[1]
        w2p = W[2][perm[Ci], :] if Ci in perm else W[2]
        w2pp = w2p[:, iperm[Co]] if Co in perm else w2p
        w4s = jnp.stack([W[0], W[2], w1p, w2pp])
        tp_dtype = jnp.bfloat16 if Co >= 32 else jnp.float32
        r1p = _sc_pass[Ci](a, *edges[Ci], zeros[Ci])
        op, tp = _tc_mid(r1p, h, dis, w4s, Ci, Co, tp_dtype)
        r2p = _sc_pass[Co](tp, *edges[Co], zeros[Co])
        if li < 3:
            a_dtype = jnp.bfloat16 if Co >= 32 else jnp.float32
            h, a = _tc_post(r2p, op, dis, b8, Co, a_dtype)
        else:
            return _tc_final(r2p, op, dis, b8)


# speedup vs baseline: 1.2944x; 1.0001x over previous
import jax
import jax.numpy as jnp
import numpy as np
from jax import lax
from jax.experimental import pallas as pl
from jax.experimental.pallas import tpu as pltpu
from jax.experimental.pallas import tpu_sc as plsc

N = 10000
E = 320000
NC = 2
NS = 16
NW = NC * NS
K = 128
NCHUNK = 80
EPT = NCHUNK * K
E_PAD = NW * EPT - E
RPT = 624
TAIL = N - NS * RPT
L = 16
BN = 1000


def _sc_mesh():
    return plsc.VectorSubcoreMesh(core_axis_name="c", subcore_axis_name="s")


def _stage_edges(src_hbm, dst_hbm, w_hbm, srcm, dstm, wm, gwid):
    base = gwid * NCHUNK
    pltpu.sync_copy(src_hbm.at[pl.ds(base, NCHUNK)], srcm)
    pltpu.sync_copy(dst_hbm.at[pl.ds(base, NCHUNK)], dstm)
    pltpu.sync_copy(w_hbm.at[pl.ds(base, NCHUNK)], wm)

    def mask_chunk(i, carry):
        for g in range(K // L):
            sl = pl.ds(g * L, L)
            s16 = srcm[i, sl]
            d16 = dstm[i, sl]
            w16 = wm[i, sl]
            wm[i, sl] = jnp.where(s16 == d16, jnp.zeros((L,), jnp.float32), w16)
        return carry

    lax.fori_loop(0, NCHUNK, mask_chunk, 0)


def _rowwise_copy(copy_fn, sid):
    copy_fn(pl.ds(sid * RPT, RPT))

    @pl.when(sid == NS - 1)
    def _():
        copy_fn(pl.ds(NS * RPT, TAIL))


def _make_sc_edge_pass(C):
    KC = 16 if C == 128 else K
    TOTCH = (NW * EPT) // KC
    NB, LEAD = 4, 2
    BF = C >= 32
    if C == 128:
        CNT0, CNT1 = 736, 544
    elif C in (64, 32):
        CNT0, CNT1 = 88, 72
    else:
        CNT0, CNT1 = 112, 48

    def body(y_hbm, src_hbm, dst_hbm, w_hbm, z_hbm, out_hbm,
             acc, srcm, dstm, wm, rows, srow, *sems):
        gsems = sems[:NB]
        ssems = sems[NB:]
        cid = lax.axis_index("c")
        sid = lax.axis_index("s")
        _rowwise_copy(lambda sl: pltpu.sync_copy(z_hbm.at[sl], acc.at[sl]), sid)
        start = jnp.where(cid == 0, sid * CNT0, NS * CNT0 + sid * CNT1)
        nch = jnp.where(cid == 0, CNT0, CNT1)

        @pl.when(cid == 0)
        def _():
            pltpu.sync_copy(src_hbm.at[pl.ds(start, CNT0)], srcm)
            pltpu.sync_copy(dst_hbm.at[pl.ds(start, CNT0)], dstm)
            pltpu.sync_copy(w_hbm.at[pl.ds(start, CNT0)], wm)

        @pl.when(cid == 1)
        def _():
            csl = pl.ds(0, CNT1)
            pltpu.sync_copy(src_hbm.at[pl.ds(start, CNT1)], srcm.at[csl])
            pltpu.sync_copy(dst_hbm.at[pl.ds(start, CNT1)], dstm.at[csl])
            pltpu.sync_copy(w_hbm.at[pl.ds(start, CNT1)], wm.at[csl])

        def mask_chunk(i, carry):
            for g in range(KC // L):
                sl = pl.ds(g * L, L)
                wm[i, sl] = jnp.where(srcm[i, sl] == dstm[i, sl],
                                      jnp.zeros((L,), jnp.float32), wm[i, sl])
            return carry

        lax.fori_loop(0, nch, mask_chunk, 0)
        plsc.subcore_barrier()

        for c0 in range(LEAD):
            pltpu.async_copy(y_hbm.at[srcm.at[c0]], rows.at[c0], gsems[c0])

        def visit(cur, b):
            pltpu.make_async_copy(
                y_hbm.at[srcm.at[cur]], rows.at[b], gsems[b]).wait()
            prev = cur - NB

            @pl.when(prev >= 0)
            def _():
                pltpu.make_async_copy(
                    srow.at[b], acc.at[dstm.at[prev]], ssems[b]).wait()

            def gbody(g, cc):
                w16 = wm[cur, pl.ds(g * L, L)]
                for j in range(L):
                    we = w16[j]
                    e = g * L + j
                    if BF:
                        for ci in range(C // (2 * L)):
                            v = rows[b, e, pl.ds(ci * 2 * L, 2 * L)]
                            lo, hi = plsc.unpack(
                                v, format=plsc.PackFormat.INTERLEAVED)
                            srow[b, e, pl.ds(ci * L, L)] = lo * we
                            srow[b, e, pl.ds(C // 2 + ci * L, L)] = hi * we
                    else:
                        for ci in range(C // L):
                            csl = pl.ds(ci * L, L)
                            srow[b, e, csl] = rows[b, e, csl] * we
                return cc

            lax.fori_loop(0, KC // L, gbody, 0)
            pltpu.async_copy(srow.at[b], acc.at[dstm.at[cur]], ssems[b],
                             add=True)
            nxt = cur + LEAD
            bj = (b + LEAD) % NB

            @pl.when(nxt < nch)
            def _():
                pltpu.async_copy(y_hbm.at[srcm.at[nxt]], rows.at[bj],
                                 gsems[bj])

        def outer(i, carry):
            for b in range(NB):
                visit(i * NB + b, b)
            return carry

        lax.fori_loop(0, nch // NB, outer, 0)
        for t in range(NB):
            pltpu.make_async_copy(
                srow.at[t], acc.at[dstm.at[nch - NB + t]],
                ssems[t]).wait()
        plsc.subcore_barrier()
        _rowwise_copy(
            lambda sl: pltpu.sync_copy(acc.at[sl], out_hbm.at[cid, sl]), sid)

    return pl.kernel(
        body,
        out_type=jax.ShapeDtypeStruct((NC, N, C), jnp.float32),
        mesh=_sc_mesh(),
        compiler_params=pltpu.CompilerParams(use_tc_tiling_on_sc=False, needs_layout_passes=False),
        scratch_types=[
            pltpu.VMEM_SHARED((N, C), jnp.float32),
            pltpu.VMEM((CNT0, KC), jnp.int32),
            pltpu.VMEM((CNT0, KC), jnp.int32),
            pltpu.VMEM((CNT0, KC), jnp.float32),
            pltpu.VMEM((NB, KC, C), jnp.bfloat16 if BF else jnp.float32),
            pltpu.VMEM((NB, KC, C), jnp.float32),
        ] + [pltpu.SemaphoreType.DMA] * (2 * NB),
    )


def _make_sc_deg():
    def body(src_hbm, dst_hbm, w_hbm, z_hbm, out_hbm,
             acc, srcm, dstm, wm, rows, sem0, sem1):
        ssems = (sem0, sem1)
        cid = lax.axis_index("c")
        sid = lax.axis_index("s")
        gwid = cid * NS + sid
        _rowwise_copy(lambda sl: pltpu.sync_copy(z_hbm.at[sl], acc.at[sl]), sid)
        _stage_edges(src_hbm, dst_hbm, w_hbm, srcm, dstm, wm, gwid)
        plsc.subcore_barrier()

        def visit(cur, b):
            @pl.when(cur >= 2)
            def _():
                pltpu.make_async_copy(
                    rows.at[b], acc.at[srcm.at[cur - 2]], ssems[b]).wait()

            def gbody(g, cc):
                w16 = wm[cur, pl.ds(g * L, L)]
                for j in range(L):
                    e = g * L + j
                    rows[b, e, pl.ds(0, L)] = jnp.full((L,), w16[j],
                                                       jnp.float32)
                return cc

            lax.fori_loop(0, K // L, gbody, 0)
            pltpu.async_copy(rows.at[b], acc.at[srcm.at[cur]], ssems[b],
                             add=True)

        def outer(i, carry):
            for b in range(2):
                visit(i * 2 + b, b)
            return carry

        lax.fori_loop(0, NCHUNK // 2, outer, 0)
        for tail in range(NCHUNK - 2, NCHUNK):
            pltpu.make_async_copy(
                rows.at[tail % 2], acc.at[srcm.at[tail]],
                ssems[tail % 2]).wait()
        plsc.subcore_barrier()
        _rowwise_copy(
            lambda sl: pltpu.sync_copy(acc.at[sl], out_hbm.at[cid, sl]), sid)

    return pl.kernel(
        body,
        out_type=jax.ShapeDtypeStruct((NC, N, L), jnp.float32),
        mesh=_sc_mesh(),
        compiler_params=pltpu.CompilerParams(use_tc_tiling_on_sc=False, needs_layout_passes=False),
        scratch_types=[
            pltpu.VMEM_SHARED((N, L), jnp.float32),
            pltpu.VMEM((NCHUNK, K), jnp.int32),
            pltpu.VMEM((NCHUNK, K), jnp.int32),
            pltpu.VMEM((NCHUNK, K), jnp.float32),
            pltpu.VMEM((2, K, L), jnp.float32),
            pltpu.SemaphoreType.DMA,
            pltpu.SemaphoreType.DMA,
        ],
    )


def _tc_pre(degp, x):
    def body(degp_ref, x_ref, dis_ref, a_ref):
        d = degp_ref[0] + degp_ref[1]
        pos = d > 0.0
        safe = jnp.where(pos, d, 1.0)
        dis = jnp.where(pos, lax.rsqrt(safe), 0.0)
        dis_ref[...] = dis
        a_ref[...] = (x_ref[...] * dis[:, 0:1]).astype(jnp.bfloat16)

    return pl.pallas_call(
        body,
        grid=(N // BN,),
        in_specs=[
            pl.BlockSpec((2, BN, L), lambda i: (0, i, 0)),
            pl.BlockSpec((BN, 128), lambda i: (i, 0)),
        ],
        out_specs=[
            pl.BlockSpec((BN, L), lambda i: (i, 0)),
            pl.BlockSpec((BN, 128), lambda i: (i, 0)),
        ],
        out_shape=[
            jax.ShapeDtypeStruct((N, L), jnp.float32),
            jax.ShapeDtypeStruct((N, 128), jnp.bfloat16),
        ],
    )(degp, x)


def _tc_mid(r1p, h, dis, W4s, Ci, Co, tp_dtype):
    def body(r_ref, h_ref, dis_ref, w_ref, op_ref, tp_ref):
        d = dis_ref[:, 0:1]
        y1 = -d * (r_ref[0] + r_ref[1])
        v0 = w_ref[0] - w_ref[1]
        op = jnp.dot(h_ref[...], v0, preferred_element_type=jnp.float32)
        op = op + jnp.dot(y1, w_ref[2], preferred_element_type=jnp.float32)
        op_ref[...] = op
        tp = d * jnp.dot(y1, w_ref[3], preferred_element_type=jnp.float32)
        tp_ref[...] = tp.astype(tp_dtype)

    return pl.pallas_call(
        body,
        grid=(N // BN,),
        in_specs=[
            pl.BlockSpec((2, BN, Ci), lambda i: (0, i, 0)),
            pl.BlockSpec((BN, Ci), lambda i: (i, 0)),
            pl.BlockSpec((BN, L), lambda i: (i, 0)),
            pl.BlockSpec((4, Ci, Co), lambda i: (0, 0, 0)),
        ],
        out_specs=[
            pl.BlockSpec((BN, Co), lambda i: (i, 0)),
            pl.BlockSpec((BN, Co), lambda i: (i, 0)),
        ],
        out_shape=[
            jax.ShapeDtypeStruct((N, Co), jnp.float32),
            jax.ShapeDtypeStruct((N, Co), tp_dtype),
        ],
    )(r1p, h, dis, W4s)


def _tc_post(r2p, op, dis, b8, Co, a_dtype):
    def body(r_ref, op_ref, dis_ref, b_ref, h_ref, a_ref):
        d = dis_ref[:, 0:1]
        o = op_ref[...] + b_ref[0:1, :] - 2.0 * d * (r_ref[0] + r_ref[1])
        h = jnp.maximum(o, 0.0)
        h_ref[...] = h
        a_ref[...] = (d * h).astype(a_dtype)

    return pl.pallas_call(
        body,
        grid=(N // BN,),
        in_specs=[
            pl.BlockSpec((2, BN, Co), lambda i: (0, i, 0)),
            pl.BlockSpec((BN, Co), lambda i: (i, 0)),
            pl.BlockSpec((BN, L), lambda i: (i, 0)),
            pl.BlockSpec((8, Co), lambda i: (0, 0)),
        ],
        out_specs=[
            pl.BlockSpec((BN, Co), lambda i: (i, 0)),
            pl.BlockSpec((BN, Co), lambda i: (i, 0)),
        ],
        out_shape=[
            jax.ShapeDtypeStruct((N, Co), jnp.float32),
            jax.ShapeDtypeStruct((N, Co), a_dtype),
        ],
    )(r2p, op, dis, b8)


def _tc_final(r2p, op, dis, b8):
    def body(r_ref, op_ref, dis_ref, b_ref, out_ref):
        d = dis_ref[:, 0:1]
        o = op_ref[...] + b_ref[0:1, :] - 2.0 * d * (r_ref[0] + r_ref[1])
        z = o[:, :10]
        m = jnp.max(z, axis=1, keepdims=True)
        zs = z - m
        lse = jnp.log(jnp.sum(jnp.exp(zs), axis=1, keepdims=True))
        out_ref[...] = zs - lse

    return pl.pallas_call(
        body,
        grid=(N // BN,),
        in_specs=[
            pl.BlockSpec((2, BN, 16), lambda i: (0, i, 0)),
            pl.BlockSpec((BN, 16), lambda i: (i, 0)),
            pl.BlockSpec((BN, L), lambda i: (i, 0)),
            pl.BlockSpec((8, 16), lambda i: (0, 0)),
        ],
        out_specs=pl.BlockSpec((BN, 10), lambda i: (i, 0)),
        out_shape=jax.ShapeDtypeStruct((N, 10), jnp.float32),
    )(r2p, op, dis, b8)


_sc_pass = {C: _make_sc_edge_pass(C) for C in (128, 64, 32, 16)}
_sc_deg = _make_sc_deg()


def kernel(x, edge_index, edge_weight, W1, b1, W2, b2, W3, b3, W4, b4):
    zpad_i = jnp.zeros((E_PAD,), jnp.int32)
    srcf = jnp.concatenate([edge_index[0], zpad_i])
    dstf = jnp.concatenate([edge_index[1], zpad_i])
    wf = jnp.concatenate([edge_weight, jnp.zeros((E_PAD,), jnp.float32)])

    def eshape(C):
        kc = 16 if C == 128 else K
        return ((NW * EPT) // kc, kc)

    edges = {C: (srcf.reshape(eshape(C)), dstf.reshape(eshape(C)),
                 wf.reshape(eshape(C)))
             for C in (128, 64, 32, 16)}
    src3, dst3, w3 = edges[16]
    zeros = {C: jnp.zeros((N, C), jnp.float32) for C in (128, 64, 32, 16)}

    W4p = jnp.zeros((3, 16, 16), jnp.float32).at[:, :, :10].set(W4)
    b4p = jnp.zeros((16,), jnp.float32).at[:10].set(b4)

    degp = _sc_deg(src3, dst3, w3, zeros[16])
    dis, a = _tc_pre(degp, x)

    layers = [
        (W1, b1, 128, 64),
        (W2, b2, 64, 32),
        (W3, b3, 32, 16),
        (W4p, b4p, 16, 16),
    ]
    perm = {c: np.concatenate([np.arange(0, c, 2), np.arange(1, c, 2)])
            for c in (128, 64, 32)}
    iperm = {c: np.argsort(perm[c]) for c in (128, 64, 32)}

    h = x
    for li, (W, b, Ci, Co) in enumerate(layers):
        b8 = jnp.broadcast_to(b, (8, Co))
        w1p = W[1][perm[Ci], :] if Ci in perm else W[1]
        w2p = W[2][perm[Ci], :] if Ci in perm else W[2]
        w2pp = w2p[:, iperm[Co]] if Co in perm else w2p
        w4s = jnp.stack([W[0], W[2], w1p, w2pp])
        tp_dtype = jnp.bfloat16 if Co >= 32 else jnp.float32
        r1p = _sc_pass[Ci](a, *edges[Ci], zeros[Ci])
        op, tp = _tc_mid(r1p, h, dis, w4s, Ci, Co, tp_dtype)
        r2p = _sc_pass[Co](tp, *edges[Co], zeros[Co])
        if li < 3:
            a_dtype = jnp.bfloat16 if Co >= 32 else jnp.float32
            h, a = _tc_post(r2p, op, dis, b8, Co, a_dtype)
        else:
            return _tc_final(r2p, op, dis, b8)
